# baseline (speedup 1.0000x reference)
import functools

import jax
import jax.numpy as jnp
from jax import lax
from jax.experimental import pallas as pl
from jax.experimental.pallas import tpu as pltpu
from jax.experimental.pallas import tpu_sc as plsc

N_NODES = 10000
NPAD = 10240
N_EDGES = 160000
EPAD = 163840
TILE = 640
NT = N_EDGES // TILE
NT_PAD = EPAD // TILE
NW = 32
PER_W = EPAD // NW
NCH = PER_W // 128
FE = float(N_EDGES)
FN = float(N_NODES)
EPS = 1e-5

MP_CFG = [(16, 10, 32, 32), (32, 32, 64, 64)]


def _lrelu(v):
    return jnp.where(v >= 0, v, 0.1 * v)


def _full(shape):
    return pl.BlockSpec(shape, lambda *_: (0,) * len(shape))


def _tiled(c):
    return pl.BlockSpec((TILE, c), lambda t: (t, 0))


def _tiled_clamped(c):
    return pl.BlockSpec((TILE, c), lambda t: (jnp.minimum(t, NT - 1), 0))


def _st(stats_ref, g_ref, b_ref, n):
    m = stats_ref[0:1, :] / n
    var = stats_ref[1:2, :] / n - m * m
    s = g_ref[...] * lax.rsqrt(var + EPS)
    return s, b_ref[...] - m * s


def _acc_stats(o_ref, v):
    t = pl.program_id(0)

    @pl.when(t == 0)
    def _():
        o_ref[...] = jnp.zeros_like(o_ref)

    o_ref[0:1, :] += jnp.sum(v, axis=0, keepdims=True)
    o_ref[1:2, :] += jnp.sum(v * v, axis=0, keepdims=True)


def _estats_body(e_ref, o_ref):
    _acc_stats(o_ref, e_ref[...])


def _estats(e):
    c = e.shape[1]
    return pl.pallas_call(
        _estats_body,
        grid=(NT,),
        in_specs=[_tiled(c)],
        out_specs=_full((2, c)),
        out_shape=jax.ShapeDtypeStruct((2, c), jnp.float32),
    )(e)


def _node_prep_body(x_ref, g_ref, b_ref, o_ref):
    v = x_ref[...]
    m = jnp.mean(v, axis=0, keepdims=True)
    var = jnp.mean(v * v, axis=0, keepdims=True) - m * m
    s = g_ref[...] * lax.rsqrt(var + EPS)
    o_ref[...] = v * s + (b_ref[...] - m * s)


def _node_prep(x, g, b):
    c = x.shape[1]
    return pl.pallas_call(
        _node_prep_body,
        grid=(1,),
        in_specs=[_full((N_NODES, c)), _full((1, c)), _full((1, c))],
        out_specs=_full((N_NODES, c)),
        out_shape=jax.ShapeDtypeStruct((N_NODES, c), jnp.float32),
    )(x, g.reshape(1, -1), b.reshape(1, -1))


def _h1(ni, src_ref, dst_ref, e_ref, es_ref, eg_ref, eb_ref,
        w1a_ref, w1b_ref, w1c_ref, b1_ref):
    se, te = _st(es_ref, eg_ref, eb_ref, FE)
    en = e_ref[...] * se + te
    h = (jnp.dot(src_ref[:, :ni], w1a_ref[...], preferred_element_type=jnp.float32)
         + jnp.dot(dst_ref[:, :ni], w1b_ref[...], preferred_element_type=jnp.float32)
         + jnp.dot(en, w1c_ref[...], preferred_element_type=jnp.float32)
         + b1_ref[...])
    return _lrelu(h), en


def _passA_body(ni, src_ref, dst_ref, e_ref, es_ref, eg_ref, eb_ref,
                w1a_ref, w1b_ref, w1c_ref, b1_ref, o_ref):
    h, _ = _h1(ni, src_ref, dst_ref, e_ref, es_ref, eg_ref, eb_ref,
               w1a_ref, w1b_ref, w1c_ref, b1_ref)
    _acc_stats(o_ref, h)


def _passB_body(ni, src_ref, dst_ref, e_ref, es_ref, eg_ref, eb_ref,
                w1a_ref, w1b_ref, w1c_ref, b1_ref,
                s1_ref, g1_ref, bb1_ref, w2_ref, b2_ref,
                lw_ref, lb_ref, h2_ref, o_ref):
    h, en = _h1(ni, src_ref, dst_ref, e_ref, es_ref, eg_ref, eb_ref,
                w1a_ref, w1b_ref, w1c_ref, b1_ref)
    s1, t1 = _st(s1_ref, g1_ref, bb1_ref, FE)
    h = h * s1 + t1
    h2 = (_lrelu(jnp.dot(h, w2_ref[...], preferred_element_type=jnp.float32)
                 + b2_ref[...])
          + jnp.dot(en, lw_ref[...], preferred_element_type=jnp.float32)
          + lb_ref[...])
    h2_ref[...] = h2
    _acc_stats(o_ref, h2)


def _passC_body(h2_ref, s2_ref, g2_ref, bb2_ref, nw1_ref, nb1_ref, o_ref):
    s2, t2 = _st(s2_ref, g2_ref, bb2_ref, FE)
    en = h2_ref[...] * s2 + t2
    w = _lrelu(jnp.dot(en, nw1_ref[...], preferred_element_type=jnp.float32)
               + nb1_ref[...])
    _acc_stats(o_ref, w)


def _passD_body(ni, no, h2_ref, src_ref, s2_ref, g2_ref, bb2_ref,
                s3_ref, g3_ref, bb3_ref, nw1_ref, nb1_ref,
                nw2_ref, nb2_ref, msg_ref):
    s2, t2 = _st(s2_ref, g2_ref, bb2_ref, FE)
    en = h2_ref[...] * s2 + t2
    w = _lrelu(jnp.dot(en, nw1_ref[...], preferred_element_type=jnp.float32)
               + nb1_ref[...])
    s3, t3 = _st(s3_ref, g3_ref, bb3_ref, FE)
    wn = w * s3 + t3
    wf = jnp.dot(wn, nw2_ref[...], preferred_element_type=jnp.float32) + nb2_ref[...]
    src = src_ref[:, :ni]
    acc = src[:, 0:1] * wf[:, 0:no]
    for i in range(1, ni):
        acc += src[:, i:i + 1] * wf[:, i * no:(i + 1) * no]
    rows = pl.program_id(0) * TILE + lax.broadcasted_iota(jnp.int32, (TILE, 1), 0)
    acc = jnp.where(rows < N_EDGES, acc, 0.0)
    msg_ref[...] = jnp.concatenate(
        [acc, jnp.zeros((TILE, 128 - no), jnp.float32)], axis=1)


def _passF_body(no, agg_ref, xn_ref, root_ref, bias_ref, g_ref, b_ref, o_ref):
    agg = (agg_ref[0:N_NODES, :no] + agg_ref[NPAD:NPAD + N_NODES, :no])
    xp = (agg
          + jnp.dot(xn_ref[...], root_ref[...], preferred_element_type=jnp.float32)
          + bias_ref[...])
    m = jnp.mean(xp, axis=0, keepdims=True)
    var = jnp.mean(xp * xp, axis=0, keepdims=True) - m * m
    s = g_ref[...] * lax.rsqrt(var + EPS)
    o_ref[...] = xp * s + (b_ref[...] - m * s)


def _edge_mlp_body(src_ref, dst_ref, e_ref, es_ref, eg_ref, eb_ref,
                   w0a_ref, w0b_ref, w0c_ref, b0_ref,
                   w1_ref, b1_ref, w2_ref, b2_ref, w3_ref, b3_ref,
                   w4_ref, b4_ref, o_ref):
    se, te = _st(es_ref, eg_ref, eb_ref, FE)
    en = e_ref[...] * se + te
    h = (jnp.dot(src_ref[:, :64], w0a_ref[...], preferred_element_type=jnp.float32)
         + jnp.dot(dst_ref[:, :64], w0b_ref[...], preferred_element_type=jnp.float32)
         + jnp.dot(en, w0c_ref[...], preferred_element_type=jnp.float32)
         + b0_ref[...])
    h = _lrelu(h)
    for wr, br, last in ((w1_ref, b1_ref, False), (w2_ref, b2_ref, False),
                         (w3_ref, b3_ref, False), (w4_ref, b4_ref, True)):
        h = jnp.dot(h, wr[...], preferred_element_type=jnp.float32) + br[...]
        if not last:
            h = _lrelu(h)
    o_ref[...] = h


def _node_mlp_body(x_ref, w0_ref, b0_ref, w1_ref, b1_ref, w2_ref, b2_ref,
                   w3_ref, b3_ref, w4_ref, b4_ref, o_ref):
    h = x_ref[...]
    for wr, br, last in ((w0_ref, b0_ref, False), (w1_ref, b1_ref, False),
                         (w2_ref, b2_ref, False), (w3_ref, b3_ref, False),
                         (w4_ref, b4_ref, True)):
        h = jnp.dot(h, wr[...], preferred_element_type=jnp.float32) + br[...]
        if not last:
            h = _lrelu(h)
    o_ref[...] = h


def _sc_gather(c):
    mesh = plsc.VectorSubcoreMesh(core_axis_name="c", subcore_axis_name="s")

    @functools.partial(
        pl.kernel,
        mesh=mesh,
        out_type=[jax.ShapeDtypeStruct((EPAD, 128), jnp.float32),
                  jax.ShapeDtypeStruct((EPAD, 128), jnp.float32)],
        scratch_types=[pltpu.VMEM((NCH, 128), jnp.int32),
                       pltpu.VMEM((NCH, 128), jnp.int32),
                       pltpu.VMEM((128, 128), jnp.float32),
                       pltpu.SemaphoreType.DMA],
    )
    def k(xp_hbm, row_hbm, col_hbm, src_hbm, dst_hbm, row_v, col_v, buf, sem):
        wid = lax.axis_index("s") * 2 + lax.axis_index("c")
        pltpu.sync_copy(row_hbm.at[pl.ds(wid * NCH, NCH)], row_v)
        pltpu.sync_copy(col_hbm.at[pl.ds(wid * NCH, NCH)], col_v)

        def body(j, carry):
            base = wid * PER_W + j * 128
            pltpu.async_copy(xp_hbm.at[row_v.at[j]], buf, sem).wait()
            pltpu.sync_copy(buf, src_hbm.at[pl.ds(base, 128)])
            pltpu.async_copy(xp_hbm.at[col_v.at[j]], buf, sem).wait()
            pltpu.sync_copy(buf, dst_hbm.at[pl.ds(base, 128)])
            return carry

        lax.fori_loop(0, NCH, body, 0)

    return k


def _sc_scatter():
    mesh = plsc.VectorSubcoreMesh(core_axis_name="c", subcore_axis_name="s")

    @functools.partial(
        pl.kernel,
        mesh=mesh,
        out_type=jax.ShapeDtypeStruct((2 * NPAD, 128), jnp.float32),
        scratch_types=[pltpu.VMEM((NCH, 128), jnp.int32),
                       pltpu.VMEM((128, 128), jnp.float32),
                       pltpu.VMEM_SHARED((NPAD, 128), jnp.float32),
                       pltpu.SemaphoreType.DMA],
    )
    def k(msg_hbm, col_hbm, zero_hbm, out_hbm, col_v, buf, acc_sh, sem):
        cid = lax.axis_index("c")
        sid = lax.axis_index("s")
        wid = sid * 2 + cid

        @pl.when(sid == 0)
        def _():
            pltpu.sync_copy(zero_hbm, acc_sh)

        plsc.subcore_barrier()
        pltpu.sync_copy(col_hbm.at[pl.ds(wid * NCH, NCH)], col_v)

        def body(j, carry):
            pltpu.sync_copy(msg_hbm.at[pl.ds(wid * PER_W + j * 128, 128)], buf)
            pltpu.sync_copy(buf, acc_sh.at[col_v.at[j]], add=True)
            return carry

        lax.fori_loop(0, NCH, body, 0)
        plsc.subcore_barrier()
        step = NPAD // 16
        pltpu.sync_copy(acc_sh.at[pl.ds(sid * step, step)],
                        out_hbm.at[pl.ds(cid * NPAD + sid * step, step)])

    return k


def _edge_pass(body, n_out_stats, tiled_in, full_in, extra_out=None,
               grid_n=NT, clamp=False):
    tspec = _tiled_clamped if clamp else _tiled
    in_specs = [tspec(w) for _, w in tiled_in]
    tiled_in = [a for a, _ in tiled_in]
    in_specs += [_full(a.shape) for a in full_in]
    out_specs = []
    out_shapes = []
    if extra_out is not None:
        out_specs.append(pl.BlockSpec((TILE, extra_out[1]), lambda t: (t, 0)))
        out_shapes.append(jax.ShapeDtypeStruct(extra_out, jnp.float32))
    if n_out_stats:
        out_specs.append(_full((2, n_out_stats)))
        out_shapes.append(jax.ShapeDtypeStruct((2, n_out_stats), jnp.float32))
    out_specs = out_specs[0] if len(out_specs) == 1 else out_specs
    out_shapes = out_shapes[0] if len(out_shapes) == 1 else out_shapes
    return pl.pallas_call(
        body,
        grid=(grid_n,),
        in_specs=in_specs,
        out_specs=out_specs,
        out_shape=out_shapes,
    )(*tiled_in, *full_in)


def kernel(x, e, params, edge_index, xbatch):
    p = params
    q = lambda v: v.reshape(1, -1)
    pad = EPAD - N_EDGES
    row2 = jnp.concatenate(
        [edge_index[0], jnp.zeros((pad,), jnp.int32)]).reshape(EPAD // 128, 128)
    col2 = jnp.concatenate(
        [edge_index[1], jnp.zeros((pad,), jnp.int32)]).reshape(EPAD // 128, 128)

    e_stats = _estats(e)
    e_cur, e_g, e_b = e, q(p['bn_edge_g']), q(p['bn_edge_b'])
    xn = _node_prep(x, p['bn_node_g'], p['bn_node_b'])

    for i, (ni, ei, no, eo) in enumerate(MP_CFG):
        pre = 'em%d' % i
        w1 = p[pre + '_W1']
        w1a, w1b, w1c = w1[:ni], w1[ni:2 * ni], w1[2 * ni:]
        nout = w1.shape[1]
        xnp = jnp.pad(xn, ((0, 0), (0, 128 - ni)))
        src, dst = _sc_gather(ni)(xnp, row2, col2)

        stats1 = _edge_pass(
            functools.partial(_passA_body, ni), nout,
            [(src, 128), (dst, 128), (e_cur, ei)],
            [e_stats, e_g, e_b, w1a, w1b, w1c, q(p[pre + '_b1'])])

        h2, stats2 = _edge_pass(
            functools.partial(_passB_body, ni), eo,
            [(src, 128), (dst, 128), (e_cur, ei)],
            [e_stats, e_g, e_b, w1a, w1b, w1c, q(p[pre + '_b1']),
             stats1, q(p[pre + '_bn1_g']), q(p[pre + '_bn1_b']),
             p[pre + '_W2'], q(p[pre + '_b2']),
             p[pre + '_linW'], q(p[pre + '_linb'])],
            extra_out=(N_EDGES, eo))

        g2, b2 = q(p[pre + '_bn2_g']), q(p[pre + '_bn2_b'])
        npre = 'nn%d' % i
        stats3 = _edge_pass(
            _passC_body, 2 * eo,
            [(h2, eo)],
            [stats2, g2, b2, p[npre + '_W1'], q(p[npre + '_b1'])])

        msg = _edge_pass(
            functools.partial(_passD_body, ni, no), 0,
            [(h2, eo), (src, 128)],
            [stats2, g2, b2, stats3,
             q(p[npre + '_bn_g']), q(p[npre + '_bn_b']),
             p[npre + '_W1'], q(p[npre + '_b1']),
             p[npre + '_W2'], q(p[npre + '_b2'])],
            extra_out=(EPAD, 128), grid_n=NT_PAD, clamp=True)

        zero = jnp.zeros((NPAD, 128), jnp.float32)
        aggp = _sc_scatter()(msg, col2, zero)

        xn = pl.pallas_call(
            functools.partial(_passF_body, no),
            grid=(1,),
            in_specs=[_full((2 * NPAD, 128)), _full((N_NODES, ni)),
                      _full((ni, no)), _full((1, no)), _full((1, no)),
                      _full((1, no))],
            out_specs=_full((N_NODES, no)),
            out_shape=jax.ShapeDtypeStruct((N_NODES, no), jnp.float32),
        )(aggp, xn, p['conv%d_root' % i], q(p['conv%d_bias' % i]),
          q(p['bn%d_g' % i]), q(p['bn%d_b' % i]))

        e_cur, e_stats, e_g, e_b = h2, stats2, g2, b2

    src, dst = _sc_gather(64)(jnp.pad(xn, ((0, 0), (0, 64))), row2, col2)
    w0 = p['ep_W0']
    he = _edge_pass(
        _edge_mlp_body, 0,
        [(src, 128), (dst, 128), (e_cur, 64)],
        [e_stats, e_g, e_b, w0[:64], w0[64:128], w0[128:], q(p['ep_b0']),
         p['ep_W1'], q(p['ep_b1']), p['ep_W2'], q(p['ep_b2']),
         p['ep_W3'], q(p['ep_b3']), p['ep_W4'], q(p['ep_b4'])],
        extra_out=(N_EDGES, 2))

    hn = pl.pallas_call(
        _node_mlp_body,
        grid=(1,),
        in_specs=[_full((N_NODES, 64))] + [
            _full(p['np_W%d' % j].shape) if k == 0 else _full((1, p['np_W%d' % j].shape[1]))
            for j in range(5) for k in range(2)],
        out_specs=_full((N_NODES, 2)),
        out_shape=jax.ShapeDtypeStruct((N_NODES, 2), jnp.float32),
    )(xn, *[a for j in range(5) for a in (p['np_W%d' % j], q(p['np_b%d' % j]))])

    return hn, he


# speedup vs baseline: 1.0501x; 1.0501x over previous
import functools

import jax
import jax.numpy as jnp
from jax import lax
from jax.experimental import pallas as pl
from jax.experimental.pallas import tpu as pltpu
from jax.experimental.pallas import tpu_sc as plsc

N_NODES = 10000
NPAD = 10240
N_EDGES = 160000
EPAD = 163840
TILE = 640
NT = N_EDGES // TILE
NT_PAD = EPAD // TILE
NW = 32
PER_W = EPAD // NW
NCH = PER_W // 128
FE = float(N_EDGES)
FN = float(N_NODES)
EPS = 1e-5

MP_CFG = [(16, 10, 32, 32), (32, 32, 64, 64)]


def _lrelu(v):
    return jnp.where(v >= 0, v, 0.1 * v)


def _full(shape):
    return pl.BlockSpec(shape, lambda *_: (0,) * len(shape))


def _tiled(c):
    return pl.BlockSpec((TILE, c), lambda t: (t, 0))


def _tiled_clamped(c):
    return pl.BlockSpec((TILE, c), lambda t: (jnp.minimum(t, NT - 1), 0))


def _st(stats_ref, g_ref, b_ref, n):
    m = stats_ref[0:1, :] / n
    var = stats_ref[1:2, :] / n - m * m
    s = g_ref[...] * lax.rsqrt(var + EPS)
    return s, b_ref[...] - m * s


def _acc_stats(o_ref, v):
    t = pl.program_id(0)

    @pl.when(t == 0)
    def _():
        o_ref[...] = jnp.zeros_like(o_ref)

    o_ref[0:1, :] += jnp.sum(v, axis=0, keepdims=True)
    o_ref[1:2, :] += jnp.sum(v * v, axis=0, keepdims=True)


def _estats_body(e_ref, o_ref):
    _acc_stats(o_ref, e_ref[...])


def _estats(e):
    c = e.shape[1]
    return pl.pallas_call(
        _estats_body,
        grid=(NT,),
        in_specs=[_tiled(c)],
        out_specs=_full((2, c)),
        out_shape=jax.ShapeDtypeStruct((2, c), jnp.float32),
    )(e)


def _node_prep_body(x_ref, g_ref, b_ref, o_ref):
    v = x_ref[...]
    m = jnp.mean(v, axis=0, keepdims=True)
    var = jnp.mean(v * v, axis=0, keepdims=True) - m * m
    s = g_ref[...] * lax.rsqrt(var + EPS)
    o_ref[...] = v * s + (b_ref[...] - m * s)


def _node_prep(x, g, b):
    c = x.shape[1]
    return pl.pallas_call(
        _node_prep_body,
        grid=(1,),
        in_specs=[_full((N_NODES, c)), _full((1, c)), _full((1, c))],
        out_specs=_full((N_NODES, c)),
        out_shape=jax.ShapeDtypeStruct((N_NODES, c), jnp.float32),
    )(x, g.reshape(1, -1), b.reshape(1, -1))


def _h1(ni, src_ref, dst_ref, e_ref, es_ref, eg_ref, eb_ref,
        w1a_ref, w1b_ref, w1c_ref, b1_ref):
    se, te = _st(es_ref, eg_ref, eb_ref, FE)
    en = e_ref[...] * se + te
    h = (jnp.dot(src_ref[:, :ni], w1a_ref[...], preferred_element_type=jnp.float32)
         + jnp.dot(dst_ref[:, :ni], w1b_ref[...], preferred_element_type=jnp.float32)
         + jnp.dot(en, w1c_ref[...], preferred_element_type=jnp.float32)
         + b1_ref[...])
    return _lrelu(h), en


def _passA_body(ni, src_ref, dst_ref, e_ref, es_ref, eg_ref, eb_ref,
                w1a_ref, w1b_ref, w1c_ref, b1_ref, o_ref):
    h, _ = _h1(ni, src_ref, dst_ref, e_ref, es_ref, eg_ref, eb_ref,
               w1a_ref, w1b_ref, w1c_ref, b1_ref)
    _acc_stats(o_ref, h)


def _passB_body(ni, src_ref, dst_ref, e_ref, es_ref, eg_ref, eb_ref,
                w1a_ref, w1b_ref, w1c_ref, b1_ref,
                s1_ref, g1_ref, bb1_ref, w2_ref, b2_ref,
                lw_ref, lb_ref, h2_ref, o_ref):
    h, en = _h1(ni, src_ref, dst_ref, e_ref, es_ref, eg_ref, eb_ref,
                w1a_ref, w1b_ref, w1c_ref, b1_ref)
    s1, t1 = _st(s1_ref, g1_ref, bb1_ref, FE)
    h = h * s1 + t1
    h2 = (_lrelu(jnp.dot(h, w2_ref[...], preferred_element_type=jnp.float32)
                 + b2_ref[...])
          + jnp.dot(en, lw_ref[...], preferred_element_type=jnp.float32)
          + lb_ref[...])
    h2_ref[...] = h2
    _acc_stats(o_ref, h2)


def _passC_body(h2_ref, s2_ref, g2_ref, bb2_ref, nw1_ref, nb1_ref, o_ref):
    s2, t2 = _st(s2_ref, g2_ref, bb2_ref, FE)
    en = h2_ref[...] * s2 + t2
    w = _lrelu(jnp.dot(en, nw1_ref[...], preferred_element_type=jnp.float32)
               + nb1_ref[...])
    _acc_stats(o_ref, w)


def _passD_body(ni, no, h2_ref, src_ref, s2_ref, g2_ref, bb2_ref,
                s3_ref, g3_ref, bb3_ref, nw1_ref, nb1_ref,
                nw2_ref, nb2_ref, msg_ref):
    s2, t2 = _st(s2_ref, g2_ref, bb2_ref, FE)
    en = h2_ref[...] * s2 + t2
    w = _lrelu(jnp.dot(en, nw1_ref[...], preferred_element_type=jnp.float32)
               + nb1_ref[...])
    s3, t3 = _st(s3_ref, g3_ref, bb3_ref, FE)
    wn = w * s3 + t3
    wf = jnp.dot(wn, nw2_ref[...], preferred_element_type=jnp.float32) + nb2_ref[...]
    src = src_ref[:, :ni]
    acc = src[:, 0:1] * wf[:, 0:no]
    for i in range(1, ni):
        acc += src[:, i:i + 1] * wf[:, i * no:(i + 1) * no]
    rows = pl.program_id(0) * TILE + lax.broadcasted_iota(jnp.int32, (TILE, 1), 0)
    acc = jnp.where(rows < N_EDGES, acc, 0.0)
    msg_ref[...] = jnp.concatenate(
        [acc, jnp.zeros((TILE, 128 - no), jnp.float32)], axis=1)


def _passF_body(no, agg_ref, xn_ref, root_ref, bias_ref, g_ref, b_ref, o_ref):
    agg = (agg_ref[0:N_NODES, :no] + agg_ref[NPAD:NPAD + N_NODES, :no])
    xp = (agg
          + jnp.dot(xn_ref[...], root_ref[...], preferred_element_type=jnp.float32)
          + bias_ref[...])
    m = jnp.mean(xp, axis=0, keepdims=True)
    var = jnp.mean(xp * xp, axis=0, keepdims=True) - m * m
    s = g_ref[...] * lax.rsqrt(var + EPS)
    o_ref[...] = xp * s + (b_ref[...] - m * s)


def _edge_mlp_body(src_ref, dst_ref, e_ref, es_ref, eg_ref, eb_ref,
                   w0a_ref, w0b_ref, w0c_ref, b0_ref,
                   w1_ref, b1_ref, w2_ref, b2_ref, w3_ref, b3_ref,
                   w4_ref, b4_ref, o_ref):
    se, te = _st(es_ref, eg_ref, eb_ref, FE)
    en = e_ref[...] * se + te
    h = (jnp.dot(src_ref[:, :64], w0a_ref[...], preferred_element_type=jnp.float32)
         + jnp.dot(dst_ref[:, :64], w0b_ref[...], preferred_element_type=jnp.float32)
         + jnp.dot(en, w0c_ref[...], preferred_element_type=jnp.float32)
         + b0_ref[...])
    h = _lrelu(h)
    for wr, br, last in ((w1_ref, b1_ref, False), (w2_ref, b2_ref, False),
                         (w3_ref, b3_ref, False), (w4_ref, b4_ref, True)):
        h = jnp.dot(h, wr[...], preferred_element_type=jnp.float32) + br[...]
        if not last:
            h = _lrelu(h)
    o_ref[...] = h


def _node_mlp_body(x_ref, w0_ref, b0_ref, w1_ref, b1_ref, w2_ref, b2_ref,
                   w3_ref, b3_ref, w4_ref, b4_ref, o_ref):
    h = x_ref[...]
    for wr, br, last in ((w0_ref, b0_ref, False), (w1_ref, b1_ref, False),
                         (w2_ref, b2_ref, False), (w3_ref, b3_ref, False),
                         (w4_ref, b4_ref, True)):
        h = jnp.dot(h, wr[...], preferred_element_type=jnp.float32) + br[...]
        if not last:
            h = _lrelu(h)
    o_ref[...] = h


def _sc_gather(c):
    mesh = plsc.VectorSubcoreMesh(core_axis_name="c", subcore_axis_name="s")

    @functools.partial(
        pl.kernel,
        mesh=mesh,
        out_type=[jax.ShapeDtypeStruct((EPAD, 128), jnp.float32),
                  jax.ShapeDtypeStruct((EPAD, 128), jnp.float32)],
        scratch_types=[pltpu.VMEM((NCH, 128), jnp.int32),
                       pltpu.VMEM((NCH, 128), jnp.int32),
                       pltpu.VMEM((128, 128), jnp.float32),
                       pltpu.VMEM((128, 128), jnp.float32),
                       pltpu.VMEM((128, 128), jnp.float32),
                       pltpu.VMEM((128, 128), jnp.float32)]
                      + [pltpu.SemaphoreType.DMA] * 8,
    )
    def k(xp_hbm, row_hbm, col_hbm, src_hbm, dst_hbm, row_v, col_v,
          bs0, bs1, bd0, bd1, gs0, gs1, gd0, gd1, ws0, ws1, wd0, wd1):
        wid = lax.axis_index("s") * 2 + lax.axis_index("c")
        pltpu.sync_copy(row_hbm.at[pl.ds(wid * NCH, NCH)], row_v)
        pltpu.sync_copy(col_hbm.at[pl.ds(wid * NCH, NCH)], col_v)

        def issue_g(j, bs, bd, gs, gd):
            pltpu.async_copy(xp_hbm.at[row_v.at[j]], bs, gs)
            pltpu.async_copy(xp_hbm.at[col_v.at[j]], bd, gd)

        def wait_g(bs, bd, gs, gd):
            pltpu.make_async_copy(xp_hbm.at[row_v.at[0]], bs, gs).wait()
            pltpu.make_async_copy(xp_hbm.at[col_v.at[0]], bd, gd).wait()

        def issue_w(j, bs, bd, ws, wd):
            base = wid * PER_W + j * 128
            pltpu.async_copy(bs, src_hbm.at[pl.ds(base, 128)], ws)
            pltpu.async_copy(bd, dst_hbm.at[pl.ds(base, 128)], wd)

        def wait_w(bs, bd, ws, wd):
            base = wid * PER_W
            pltpu.make_async_copy(bs, src_hbm.at[pl.ds(base, 128)], ws).wait()
            pltpu.make_async_copy(bd, dst_hbm.at[pl.ds(base, 128)], wd).wait()

        issue_g(0, bs0, bd0, gs0, gd0)

        def body(i, carry):
            j0 = i * 2

            @pl.when(i > 0)
            def _():
                wait_w(bs1, bd1, ws1, wd1)

            issue_g(j0 + 1, bs1, bd1, gs1, gd1)
            wait_g(bs0, bd0, gs0, gd0)
            issue_w(j0, bs0, bd0, ws0, wd0)

            wait_w(bs0, bd0, ws0, wd0)
            issue_g(jnp.minimum(j0 + 2, NCH - 1), bs0, bd0, gs0, gd0)
            wait_g(bs1, bd1, gs1, gd1)
            issue_w(j0 + 1, bs1, bd1, ws1, wd1)
            return carry

        lax.fori_loop(0, NCH // 2, body, 0)
        wait_g(bs0, bd0, gs0, gd0)
        wait_w(bs1, bd1, ws1, wd1)

    return k


def _sc_scatter():
    mesh = plsc.VectorSubcoreMesh(core_axis_name="c", subcore_axis_name="s")

    @functools.partial(
        pl.kernel,
        mesh=mesh,
        out_type=jax.ShapeDtypeStruct((2 * NPAD, 128), jnp.float32),
        scratch_types=[pltpu.VMEM((NCH, 128), jnp.int32),
                       pltpu.VMEM((128, 128), jnp.float32),
                       pltpu.VMEM((128, 128), jnp.float32),
                       pltpu.VMEM_SHARED((NPAD, 128), jnp.float32)]
                      + [pltpu.SemaphoreType.DMA] * 4,
    )
    def k(msg_hbm, col_hbm, zero_hbm, out_hbm, col_v, mb0, mb1, acc_sh,
          ls0, ls1, ss0, ss1):
        cid = lax.axis_index("c")
        sid = lax.axis_index("s")
        wid = sid * 2 + cid

        @pl.when(sid == 0)
        def _():
            pltpu.sync_copy(zero_hbm, acc_sh)

        plsc.subcore_barrier()
        pltpu.sync_copy(col_hbm.at[pl.ds(wid * NCH, NCH)], col_v)

        def issue_l(j, mb, ls):
            pltpu.async_copy(msg_hbm.at[pl.ds(wid * PER_W + j * 128, 128)], mb, ls)

        def wait_l(mb, ls):
            pltpu.make_async_copy(msg_hbm.at[pl.ds(0, 128)], mb, ls).wait()

        def issue_s(j, mb, ss):
            pltpu.async_copy(mb, acc_sh.at[col_v.at[j]], ss, add=True)

        def wait_s(mb, ss):
            pltpu.make_async_copy(mb, acc_sh.at[col_v.at[0]], ss).wait()

        issue_l(0, mb0, ls0)

        def body(i, carry):
            j0 = i * 2

            @pl.when(i > 0)
            def _():
                wait_s(mb1, ss1)

            issue_l(j0 + 1, mb1, ls1)
            wait_l(mb0, ls0)
            issue_s(j0, mb0, ss0)

            wait_s(mb0, ss0)
            issue_l(jnp.minimum(j0 + 2, NCH - 1), mb0, ls0)
            wait_l(mb1, ls1)
            issue_s(j0 + 1, mb1, ss1)
            return carry

        lax.fori_loop(0, NCH // 2, body, 0)
        wait_l(mb0, ls0)
        wait_s(mb1, ss1)
        plsc.subcore_barrier()
        step = NPAD // 16
        pltpu.sync_copy(acc_sh.at[pl.ds(sid * step, step)],
                        out_hbm.at[pl.ds(cid * NPAD + sid * step, step)])

    return k


def _edge_pass(body, n_out_stats, tiled_in, full_in, extra_out=None,
               grid_n=NT, clamp=False):
    tspec = _tiled_clamped if clamp else _tiled
    in_specs = [tspec(w) for _, w in tiled_in]
    tiled_in = [a for a, _ in tiled_in]
    in_specs += [_full(a.shape) for a in full_in]
    out_specs = []
    out_shapes = []
    if extra_out is not None:
        out_specs.append(pl.BlockSpec((TILE, extra_out[1]), lambda t: (t, 0)))
        out_shapes.append(jax.ShapeDtypeStruct(extra_out, jnp.float32))
    if n_out_stats:
        out_specs.append(_full((2, n_out_stats)))
        out_shapes.append(jax.ShapeDtypeStruct((2, n_out_stats), jnp.float32))
    out_specs = out_specs[0] if len(out_specs) == 1 else out_specs
    out_shapes = out_shapes[0] if len(out_shapes) == 1 else out_shapes
    return pl.pallas_call(
        body,
        grid=(grid_n,),
        in_specs=in_specs,
        out_specs=out_specs,
        out_shape=out_shapes,
    )(*tiled_in, *full_in)


def kernel(x, e, params, edge_index, xbatch):
    p = params
    q = lambda v: v.reshape(1, -1)
    pad = EPAD - N_EDGES
    row2 = jnp.concatenate(
        [edge_index[0], jnp.zeros((pad,), jnp.int32)]).reshape(EPAD // 128, 128)
    col2 = jnp.concatenate(
        [edge_index[1], jnp.zeros((pad,), jnp.int32)]).reshape(EPAD // 128, 128)

    e_stats = _estats(e)
    e_cur, e_g, e_b = e, q(p['bn_edge_g']), q(p['bn_edge_b'])
    xn = _node_prep(x, p['bn_node_g'], p['bn_node_b'])

    for i, (ni, ei, no, eo) in enumerate(MP_CFG):
        pre = 'em%d' % i
        w1 = p[pre + '_W1']
        w1a, w1b, w1c = w1[:ni], w1[ni:2 * ni], w1[2 * ni:]
        nout = w1.shape[1]
        xnp = jnp.pad(xn, ((0, 0), (0, 128 - ni)))
        src, dst = _sc_gather(ni)(xnp, row2, col2)

        stats1 = _edge_pass(
            functools.partial(_passA_body, ni), nout,
            [(src, 128), (dst, 128), (e_cur, ei)],
            [e_stats, e_g, e_b, w1a, w1b, w1c, q(p[pre + '_b1'])])

        h2, stats2 = _edge_pass(
            functools.partial(_passB_body, ni), eo,
            [(src, 128), (dst, 128), (e_cur, ei)],
            [e_stats, e_g, e_b, w1a, w1b, w1c, q(p[pre + '_b1']),
             stats1, q(p[pre + '_bn1_g']), q(p[pre + '_bn1_b']),
             p[pre + '_W2'], q(p[pre + '_b2']),
             p[pre + '_linW'], q(p[pre + '_linb'])],
            extra_out=(N_EDGES, eo))

        g2, b2 = q(p[pre + '_bn2_g']), q(p[pre + '_bn2_b'])
        npre = 'nn%d' % i
        stats3 = _edge_pass(
            _passC_body, 2 * eo,
            [(h2, eo)],
            [stats2, g2, b2, p[npre + '_W1'], q(p[npre + '_b1'])])

        msg = _edge_pass(
            functools.partial(_passD_body, ni, no), 0,
            [(h2, eo), (src, 128)],
            [stats2, g2, b2, stats3,
             q(p[npre + '_bn_g']), q(p[npre + '_bn_b']),
             p[npre + '_W1'], q(p[npre + '_b1']),
             p[npre + '_W2'], q(p[npre + '_b2'])],
            extra_out=(EPAD, 128), grid_n=NT_PAD, clamp=True)

        zero = jnp.zeros((NPAD, 128), jnp.float32)
        aggp = _sc_scatter()(msg, col2, zero)

        xn = pl.pallas_call(
            functools.partial(_passF_body, no),
            grid=(1,),
            in_specs=[_full((2 * NPAD, 128)), _full((N_NODES, ni)),
                      _full((ni, no)), _full((1, no)), _full((1, no)),
                      _full((1, no))],
            out_specs=_full((N_NODES, no)),
            out_shape=jax.ShapeDtypeStruct((N_NODES, no), jnp.float32),
        )(aggp, xn, p['conv%d_root' % i], q(p['conv%d_bias' % i]),
          q(p['bn%d_g' % i]), q(p['bn%d_b' % i]))

        e_cur, e_stats, e_g, e_b = h2, stats2, g2, b2

    src, dst = _sc_gather(64)(jnp.pad(xn, ((0, 0), (0, 64))), row2, col2)
    w0 = p['ep_W0']
    he = _edge_pass(
        _edge_mlp_body, 0,
        [(src, 128), (dst, 128), (e_cur, 64)],
        [e_stats, e_g, e_b, w0[:64], w0[64:128], w0[128:], q(p['ep_b0']),
         p['ep_W1'], q(p['ep_b1']), p['ep_W2'], q(p['ep_b2']),
         p['ep_W3'], q(p['ep_b3']), p['ep_W4'], q(p['ep_b4'])],
        extra_out=(N_EDGES, 2))

    hn = pl.pallas_call(
        _node_mlp_body,
        grid=(1,),
        in_specs=[_full((N_NODES, 64))] + [
            _full(p['np_W%d' % j].shape) if k == 0 else _full((1, p['np_W%d' % j].shape[1]))
            for j in range(5) for k in range(2)],
        out_specs=_full((N_NODES, 2)),
        out_shape=jax.ShapeDtypeStruct((N_NODES, 2), jnp.float32),
    )(xn, *[a for j in range(5) for a in (p['np_W%d' % j], q(p['np_b%d' % j]))])

    return hn, he


# speedup vs baseline: 1.0970x; 1.0447x over previous
import functools

import jax
import jax.numpy as jnp
from jax import lax
from jax.experimental import pallas as pl
from jax.experimental.pallas import tpu as pltpu
from jax.experimental.pallas import tpu_sc as plsc

N_NODES = 10000
NPAD = 10240
N_EDGES = 160000
EPAD = 163840
TILE = 640
NT = N_EDGES // TILE
NT_PAD = EPAD // TILE
NW = 32
PER_W = EPAD // NW
NCH = PER_W // 128
FE = float(N_EDGES)
FN = float(N_NODES)
EPS = 1e-5

MP_CFG = [(16, 10, 32, 32), (32, 32, 64, 64)]


def _lrelu(v):
    return jnp.where(v >= 0, v, 0.1 * v)


def _full(shape):
    return pl.BlockSpec(shape, lambda *_: (0,) * len(shape))


def _tiled(c):
    return pl.BlockSpec((TILE, c), lambda t: (t, 0))


def _tiled_clamped(c):
    return pl.BlockSpec((TILE, c), lambda t: (jnp.minimum(t, NT - 1), 0))


def _st(stats_ref, g_ref, b_ref, n):
    m = stats_ref[0:1, :] / n
    var = stats_ref[1:2, :] / n - m * m
    s = g_ref[...] * lax.rsqrt(var + EPS)
    return s, b_ref[...] - m * s


def _acc_stats(o_ref, v):
    t = pl.program_id(0)

    @pl.when(t == 0)
    def _():
        o_ref[...] = jnp.zeros_like(o_ref)

    o_ref[0:1, :] += jnp.sum(v, axis=0, keepdims=True)
    o_ref[1:2, :] += jnp.sum(v * v, axis=0, keepdims=True)


def _estats_body(e_ref, o_ref):
    _acc_stats(o_ref, e_ref[...])


def _estats(e):
    c = e.shape[1]
    return pl.pallas_call(
        _estats_body,
        grid=(NT,),
        in_specs=[_tiled(c)],
        out_specs=_full((2, c)),
        out_shape=jax.ShapeDtypeStruct((2, c), jnp.float32),
    )(e)


def _node_prep_body(x_ref, g_ref, b_ref, o_ref):
    v = x_ref[...]
    m = jnp.mean(v, axis=0, keepdims=True)
    var = jnp.mean(v * v, axis=0, keepdims=True) - m * m
    s = g_ref[...] * lax.rsqrt(var + EPS)
    o_ref[...] = v * s + (b_ref[...] - m * s)


def _node_prep(x, g, b):
    c = x.shape[1]
    return pl.pallas_call(
        _node_prep_body,
        grid=(1,),
        in_specs=[_full((N_NODES, c)), _full((1, c)), _full((1, c))],
        out_specs=_full((N_NODES, c)),
        out_shape=jax.ShapeDtypeStruct((N_NODES, c), jnp.float32),
    )(x, g.reshape(1, -1), b.reshape(1, -1))


def _h1(ni, src_ref, dst_ref, e_ref, es_ref, eg_ref, eb_ref,
        w1a_ref, w1b_ref, w1c_ref, b1_ref):
    se, te = _st(es_ref, eg_ref, eb_ref, FE)
    en = e_ref[...] * se + te
    h = (jnp.dot(src_ref[:, :ni], w1a_ref[...], preferred_element_type=jnp.float32)
         + jnp.dot(dst_ref[:, :ni], w1b_ref[...], preferred_element_type=jnp.float32)
         + jnp.dot(en, w1c_ref[...], preferred_element_type=jnp.float32)
         + b1_ref[...])
    return _lrelu(h), en


def _passA_body(ni, src_ref, dst_ref, e_ref, es_ref, eg_ref, eb_ref,
                w1a_ref, w1b_ref, w1c_ref, b1_ref, h_ref, o_ref):
    h, _ = _h1(ni, src_ref, dst_ref, e_ref, es_ref, eg_ref, eb_ref,
               w1a_ref, w1b_ref, w1c_ref, b1_ref)
    h_ref[...] = h
    _acc_stats(o_ref, h)


def _passB_body(h1_ref, e_ref, es_ref, eg_ref, eb_ref,
                s1_ref, g1_ref, bb1_ref, w2_ref, b2_ref,
                lw_ref, lb_ref, h2_ref, o_ref):
    se, te = _st(es_ref, eg_ref, eb_ref, FE)
    en = e_ref[...] * se + te
    s1, t1 = _st(s1_ref, g1_ref, bb1_ref, FE)
    h = h1_ref[...] * s1 + t1
    h2 = (_lrelu(jnp.dot(h, w2_ref[...], preferred_element_type=jnp.float32)
                 + b2_ref[...])
          + jnp.dot(en, lw_ref[...], preferred_element_type=jnp.float32)
          + lb_ref[...])
    h2_ref[...] = h2
    _acc_stats(o_ref, h2)


def _passC_body(h2_ref, s2_ref, g2_ref, bb2_ref, nw1_ref, nb1_ref, o_ref):
    s2, t2 = _st(s2_ref, g2_ref, bb2_ref, FE)
    en = h2_ref[...] * s2 + t2
    w = _lrelu(jnp.dot(en, nw1_ref[...], preferred_element_type=jnp.float32)
               + nb1_ref[...])
    _acc_stats(o_ref, w)


def _passD_body(ni, no, h2_ref, src_ref, s2_ref, g2_ref, bb2_ref,
                s3_ref, g3_ref, bb3_ref, nw1_ref, nb1_ref,
                nw2_ref, nb2_ref, msg_ref):
    s2, t2 = _st(s2_ref, g2_ref, bb2_ref, FE)
    en = h2_ref[...] * s2 + t2
    w = _lrelu(jnp.dot(en, nw1_ref[...], preferred_element_type=jnp.float32)
               + nb1_ref[...])
    s3, t3 = _st(s3_ref, g3_ref, bb3_ref, FE)
    wn = w * s3 + t3
    wf = jnp.dot(wn, nw2_ref[...], preferred_element_type=jnp.float32) + nb2_ref[...]
    src = src_ref[:, :ni]
    acc = src[:, 0:1] * wf[:, 0:no]
    for i in range(1, ni):
        acc += src[:, i:i + 1] * wf[:, i * no:(i + 1) * no]
    rows = pl.program_id(0) * TILE + lax.broadcasted_iota(jnp.int32, (TILE, 1), 0)
    acc = jnp.where(rows < N_EDGES, acc, 0.0)
    msg_ref[...] = jnp.concatenate(
        [acc, jnp.zeros((TILE, 128 - no), jnp.float32)], axis=1)


def _passF_body(no, agg_ref, xn_ref, root_ref, bias_ref, g_ref, b_ref, o_ref):
    agg = (agg_ref[0:N_NODES, :no] + agg_ref[NPAD:NPAD + N_NODES, :no])
    xp = (agg
          + jnp.dot(xn_ref[...], root_ref[...], preferred_element_type=jnp.float32)
          + bias_ref[...])
    m = jnp.mean(xp, axis=0, keepdims=True)
    var = jnp.mean(xp * xp, axis=0, keepdims=True) - m * m
    s = g_ref[...] * lax.rsqrt(var + EPS)
    o_ref[...] = xp * s + (b_ref[...] - m * s)


def _edge_mlp_body(src_ref, dst_ref, e_ref, es_ref, eg_ref, eb_ref,
                   w0a_ref, w0b_ref, w0c_ref, b0_ref,
                   w1_ref, b1_ref, w2_ref, b2_ref, w3_ref, b3_ref,
                   w4_ref, b4_ref, o_ref):
    se, te = _st(es_ref, eg_ref, eb_ref, FE)
    en = e_ref[...] * se + te
    h = (jnp.dot(src_ref[:, :64], w0a_ref[...], preferred_element_type=jnp.float32)
         + jnp.dot(dst_ref[:, :64], w0b_ref[...], preferred_element_type=jnp.float32)
         + jnp.dot(en, w0c_ref[...], preferred_element_type=jnp.float32)
         + b0_ref[...])
    h = _lrelu(h)
    for wr, br, last in ((w1_ref, b1_ref, False), (w2_ref, b2_ref, False),
                         (w3_ref, b3_ref, False), (w4_ref, b4_ref, True)):
        h = jnp.dot(h, wr[...], preferred_element_type=jnp.float32) + br[...]
        if not last:
            h = _lrelu(h)
    o_ref[...] = h


def _node_mlp_body(x_ref, w0_ref, b0_ref, w1_ref, b1_ref, w2_ref, b2_ref,
                   w3_ref, b3_ref, w4_ref, b4_ref, o_ref):
    h = x_ref[...]
    for wr, br, last in ((w0_ref, b0_ref, False), (w1_ref, b1_ref, False),
                         (w2_ref, b2_ref, False), (w3_ref, b3_ref, False),
                         (w4_ref, b4_ref, True)):
        h = jnp.dot(h, wr[...], preferred_element_type=jnp.float32) + br[...]
        if not last:
            h = _lrelu(h)
    o_ref[...] = h


def _sc_gather(c):
    mesh = plsc.VectorSubcoreMesh(core_axis_name="c", subcore_axis_name="s")

    @functools.partial(
        pl.kernel,
        mesh=mesh,
        out_type=[jax.ShapeDtypeStruct((EPAD, 128), jnp.float32),
                  jax.ShapeDtypeStruct((EPAD, 128), jnp.float32)],
        scratch_types=[pltpu.VMEM((NCH, 128), jnp.int32),
                       pltpu.VMEM((NCH, 128), jnp.int32),
                       pltpu.VMEM((128, 128), jnp.float32),
                       pltpu.VMEM((128, 128), jnp.float32),
                       pltpu.VMEM((128, 128), jnp.float32),
                       pltpu.VMEM((128, 128), jnp.float32)]
                      + [pltpu.SemaphoreType.DMA] * 8,
    )
    def k(xp_hbm, row_hbm, col_hbm, src_hbm, dst_hbm, row_v, col_v,
          bs0, bs1, bd0, bd1, gs0, gs1, gd0, gd1, ws0, ws1, wd0, wd1):
        wid = lax.axis_index("s") * 2 + lax.axis_index("c")
        pltpu.sync_copy(row_hbm.at[pl.ds(wid * NCH, NCH)], row_v)
        pltpu.sync_copy(col_hbm.at[pl.ds(wid * NCH, NCH)], col_v)

        def issue_g(j, bs, bd, gs, gd):
            pltpu.async_copy(xp_hbm.at[row_v.at[j]], bs, gs)
            pltpu.async_copy(xp_hbm.at[col_v.at[j]], bd, gd)

        def wait_g(bs, bd, gs, gd):
            pltpu.make_async_copy(xp_hbm.at[row_v.at[0]], bs, gs).wait()
            pltpu.make_async_copy(xp_hbm.at[col_v.at[0]], bd, gd).wait()

        def issue_w(j, bs, bd, ws, wd):
            base = wid * PER_W + j * 128
            pltpu.async_copy(bs, src_hbm.at[pl.ds(base, 128)], ws)
            pltpu.async_copy(bd, dst_hbm.at[pl.ds(base, 128)], wd)

        def wait_w(bs, bd, ws, wd):
            base = wid * PER_W
            pltpu.make_async_copy(bs, src_hbm.at[pl.ds(base, 128)], ws).wait()
            pltpu.make_async_copy(bd, dst_hbm.at[pl.ds(base, 128)], wd).wait()

        issue_g(0, bs0, bd0, gs0, gd0)

        def body(i, carry):
            j0 = i * 2

            @pl.when(i > 0)
            def _():
                wait_w(bs1, bd1, ws1, wd1)

            issue_g(j0 + 1, bs1, bd1, gs1, gd1)
            wait_g(bs0, bd0, gs0, gd0)
            issue_w(j0, bs0, bd0, ws0, wd0)

            wait_w(bs0, bd0, ws0, wd0)
            issue_g(jnp.minimum(j0 + 2, NCH - 1), bs0, bd0, gs0, gd0)
            wait_g(bs1, bd1, gs1, gd1)
            issue_w(j0 + 1, bs1, bd1, ws1, wd1)
            return carry

        lax.fori_loop(0, NCH // 2, body, 0)
        wait_g(bs0, bd0, gs0, gd0)
        wait_w(bs1, bd1, ws1, wd1)

    return k


def _sc_scatter():
    mesh = plsc.VectorSubcoreMesh(core_axis_name="c", subcore_axis_name="s")

    @functools.partial(
        pl.kernel,
        mesh=mesh,
        out_type=jax.ShapeDtypeStruct((2 * NPAD, 128), jnp.float32),
        scratch_types=[pltpu.VMEM((NCH, 128), jnp.int32),
                       pltpu.VMEM((128, 128), jnp.float32),
                       pltpu.VMEM((128, 128), jnp.float32),
                       pltpu.VMEM_SHARED((NPAD, 128), jnp.float32)]
                      + [pltpu.SemaphoreType.DMA] * 4,
    )
    def k(msg_hbm, col_hbm, zero_hbm, out_hbm, col_v, mb0, mb1, acc_sh,
          ls0, ls1, ss0, ss1):
        cid = lax.axis_index("c")
        sid = lax.axis_index("s")
        wid = sid * 2 + cid

        @pl.when(sid == 0)
        def _():
            pltpu.sync_copy(zero_hbm, acc_sh)

        plsc.subcore_barrier()
        pltpu.sync_copy(col_hbm.at[pl.ds(wid * NCH, NCH)], col_v)

        def issue_l(j, mb, ls):
            pltpu.async_copy(msg_hbm.at[pl.ds(wid * PER_W + j * 128, 128)], mb, ls)

        def wait_l(mb, ls):
            pltpu.make_async_copy(msg_hbm.at[pl.ds(0, 128)], mb, ls).wait()

        def issue_s(j, mb, ss):
            pltpu.async_copy(mb, acc_sh.at[col_v.at[j]], ss, add=True)

        def wait_s(mb, ss):
            pltpu.make_async_copy(mb, acc_sh.at[col_v.at[0]], ss).wait()

        issue_l(0, mb0, ls0)

        def body(i, carry):
            j0 = i * 2

            @pl.when(i > 0)
            def _():
                wait_s(mb1, ss1)

            issue_l(j0 + 1, mb1, ls1)
            wait_l(mb0, ls0)
            issue_s(j0, mb0, ss0)

            wait_s(mb0, ss0)
            issue_l(jnp.minimum(j0 + 2, NCH - 1), mb0, ls0)
            wait_l(mb1, ls1)
            issue_s(j0 + 1, mb1, ss1)
            return carry

        lax.fori_loop(0, NCH // 2, body, 0)
        wait_l(mb0, ls0)
        wait_s(mb1, ss1)
        plsc.subcore_barrier()
        step = NPAD // 16
        pltpu.sync_copy(acc_sh.at[pl.ds(sid * step, step)],
                        out_hbm.at[pl.ds(cid * NPAD + sid * step, step)])

    return k


def _edge_pass(body, n_out_stats, tiled_in, full_in, extra_out=None,
               grid_n=NT, clamp=False):
    tspec = _tiled_clamped if clamp else _tiled
    in_specs = [tspec(w) for _, w in tiled_in]
    tiled_in = [a for a, _ in tiled_in]
    in_specs += [_full(a.shape) for a in full_in]
    out_specs = []
    out_shapes = []
    if extra_out is not None:
        out_specs.append(pl.BlockSpec((TILE, extra_out[1]), lambda t: (t, 0)))
        out_shapes.append(jax.ShapeDtypeStruct(extra_out, jnp.float32))
    if n_out_stats:
        out_specs.append(_full((2, n_out_stats)))
        out_shapes.append(jax.ShapeDtypeStruct((2, n_out_stats), jnp.float32))
    out_specs = out_specs[0] if len(out_specs) == 1 else out_specs
    out_shapes = out_shapes[0] if len(out_shapes) == 1 else out_shapes
    return pl.pallas_call(
        body,
        grid=(grid_n,),
        in_specs=in_specs,
        out_specs=out_specs,
        out_shape=out_shapes,
    )(*tiled_in, *full_in)


def kernel(x, e, params, edge_index, xbatch):
    p = params
    q = lambda v: v.reshape(1, -1)
    pad = EPAD - N_EDGES
    row2 = jnp.concatenate(
        [edge_index[0], jnp.zeros((pad,), jnp.int32)]).reshape(EPAD // 128, 128)
    col2 = jnp.concatenate(
        [edge_index[1], jnp.zeros((pad,), jnp.int32)]).reshape(EPAD // 128, 128)

    e_stats = _estats(e)
    e_cur, e_g, e_b = e, q(p['bn_edge_g']), q(p['bn_edge_b'])
    xn = _node_prep(x, p['bn_node_g'], p['bn_node_b'])

    for i, (ni, ei, no, eo) in enumerate(MP_CFG):
        pre = 'em%d' % i
        w1 = p[pre + '_W1']
        w1a, w1b, w1c = w1[:ni], w1[ni:2 * ni], w1[2 * ni:]
        nout = w1.shape[1]
        xnp = jnp.pad(xn, ((0, 0), (0, 128 - ni)))
        src, dst = _sc_gather(ni)(xnp, row2, col2)

        h1, stats1 = _edge_pass(
            functools.partial(_passA_body, ni), nout,
            [(src, 128), (dst, 128), (e_cur, ei)],
            [e_stats, e_g, e_b, w1a, w1b, w1c, q(p[pre + '_b1'])],
            extra_out=(N_EDGES, nout))

        h2, stats2 = _edge_pass(
            _passB_body, eo,
            [(h1, nout), (e_cur, ei)],
            [e_stats, e_g, e_b,
             stats1, q(p[pre + '_bn1_g']), q(p[pre + '_bn1_b']),
             p[pre + '_W2'], q(p[pre + '_b2']),
             p[pre + '_linW'], q(p[pre + '_linb'])],
            extra_out=(N_EDGES, eo))

        g2, b2 = q(p[pre + '_bn2_g']), q(p[pre + '_bn2_b'])
        npre = 'nn%d' % i
        stats3 = _edge_pass(
            _passC_body, 2 * eo,
            [(h2, eo)],
            [stats2, g2, b2, p[npre + '_W1'], q(p[npre + '_b1'])])

        msg = _edge_pass(
            functools.partial(_passD_body, ni, no), 0,
            [(h2, eo), (src, 128)],
            [stats2, g2, b2, stats3,
             q(p[npre + '_bn_g']), q(p[npre + '_bn_b']),
             p[npre + '_W1'], q(p[npre + '_b1']),
             p[npre + '_W2'], q(p[npre + '_b2'])],
            extra_out=(EPAD, 128), grid_n=NT_PAD, clamp=True)

        zero = jnp.zeros((NPAD, 128), jnp.float32)
        aggp = _sc_scatter()(msg, col2, zero)

        xn = pl.pallas_call(
            functools.partial(_passF_body, no),
            grid=(1,),
            in_specs=[_full((2 * NPAD, 128)), _full((N_NODES, ni)),
                      _full((ni, no)), _full((1, no)), _full((1, no)),
                      _full((1, no))],
            out_specs=_full((N_NODES, no)),
            out_shape=jax.ShapeDtypeStruct((N_NODES, no), jnp.float32),
        )(aggp, xn, p['conv%d_root' % i], q(p['conv%d_bias' % i]),
          q(p['bn%d_g' % i]), q(p['bn%d_b' % i]))

        e_cur, e_stats, e_g, e_b = h2, stats2, g2, b2

    src, dst = _sc_gather(64)(jnp.pad(xn, ((0, 0), (0, 64))), row2, col2)
    w0 = p['ep_W0']
    he = _edge_pass(
        _edge_mlp_body, 0,
        [(src, 128), (dst, 128), (e_cur, 64)],
        [e_stats, e_g, e_b, w0[:64], w0[64:128], w0[128:], q(p['ep_b0']),
         p['ep_W1'], q(p['ep_b1']), p['ep_W2'], q(p['ep_b2']),
         p['ep_W3'], q(p['ep_b3']), p['ep_W4'], q(p['ep_b4'])],
        extra_out=(N_EDGES, 2))

    hn = pl.pallas_call(
        _node_mlp_body,
        grid=(1,),
        in_specs=[_full((N_NODES, 64))] + [
            _full(p['np_W%d' % j].shape) if k == 0 else _full((1, p['np_W%d' % j].shape[1]))
            for j in range(5) for k in range(2)],
        out_specs=_full((N_NODES, 2)),
        out_shape=jax.ShapeDtypeStruct((N_NODES, 2), jnp.float32),
    )(xn, *[a for j in range(5) for a in (p['np_W%d' % j], q(p['np_b%d' % j]))])

    return hn, he
